# Initial kernel scaffold; baseline (speedup 1.0000x reference)
import jax
import jax.numpy as jnp
from jax.experimental import pallas as pl


def kernel(x_, edge_index, edge_attr, question_embeddings, subgraph_mask, action_mask, action_bias, params):
    raise NotImplementedError("write your pallas kernel here")


# speedup vs baseline: 11.2662x; 11.2662x over previous
import functools

import jax
import jax.numpy as jnp
from jax import lax
from jax.experimental import pallas as pl
from jax.experimental.pallas import tpu as pltpu
from jax.experimental.pallas import tpu_sc as plsc

_ZR = 0.8
_NC = 2
_NS = 16
_NW = _NC * _NS
_CH = 96


def _relu(x):
    return jnp.maximum(x, 0.0)


def _tc_prep(x_, qe, nwt, nb, qwt, qb, mxt, mqt, mb):
    n, d = x_.shape

    def body(x_ref, qe_ref, nwt_ref, nb_ref, qwt_ref, qb_ref, mxt_ref,
             mqt_ref, mb_ref, out_ref):
        x = _relu(jnp.dot(x_ref[...], nwt_ref[...],
                          preferred_element_type=jnp.float32) + nb_ref[...])
        q = _relu(jnp.dot(qe_ref[...], qwt_ref[...],
                          preferred_element_type=jnp.float32) + qb_ref[...])
        qc = jnp.dot(q, mqt_ref[...], preferred_element_type=jnp.float32)
        out_ref[...] = _relu(
            jnp.dot(x, mxt_ref[...], preferred_element_type=jnp.float32)
            + qc + mb_ref[...])

    return pl.pallas_call(
        body,
        out_shape=jax.ShapeDtypeStruct((n, 128), jnp.float32),
    )(x_, qe, nwt, nb, qwt, qb, mxt, mqt, mb)


def _tc_pre(x, mf, t1t, t1b, t0t, t0b, gt, a_s, a_d):
    n = x.shape[0]

    def body(x_ref, m_ref, t1t_ref, t1b_ref, t0t_ref, t0b_ref, gt_ref,
             as_ref, ad_ref, h_out, asv_out, adv_out):
        x = x_ref[...]
        t1 = _relu(jnp.dot(x, t1t_ref[...],
                           preferred_element_type=jnp.float32) + t1b_ref[...])
        t0 = _relu(jnp.dot(x, t0t_ref[...],
                           preferred_element_type=jnp.float32) + t0b_ref[...])
        m = m_ref[...]
        xm = m * (_ZR * t1 + (1 - _ZR) * t0) + (1 - m) * (_ZR * t0 + (1 - _ZR) * t1)
        h = jnp.dot(xm, gt_ref[...], preferred_element_type=jnp.float32)
        h_out[...] = h
        asv_out[...] = jnp.dot(h, as_ref[...], preferred_element_type=jnp.float32)
        adv_out[...] = jnp.dot(h, ad_ref[...], preferred_element_type=jnp.float32)

    return pl.pallas_call(
        body,
        out_shape=(
            jax.ShapeDtypeStruct((n, 128), jnp.float32),
            jax.ShapeDtypeStruct((n, 1), jnp.float32),
            jax.ShapeDtypeStruct((n, 1), jnp.float32),
        ),
    )(x, mf, t1t, t1b, t0t, t0b, gt, a_s, a_d)


def _sc_gat(h, asv, adv, src, dst):
    n = h.shape[0]
    e = src.shape[0]
    epw = e // _NW
    nfull = epw // _CH
    rem = epw - nfull * _CH
    npad = _NS * 640
    nd = npad // 128
    assert n <= npad
    mesh = plsc.VectorSubcoreMesh(core_axis_name="c", subcore_axis_name="s")

    @functools.partial(
        pl.kernel,
        out_type=(
            jax.ShapeDtypeStruct((_NC, npad, 128), jnp.float32),
            jax.ShapeDtypeStruct((_NC, nd, 128), jnp.float32),
        ),
        mesh=mesh,
        compiler_params=pltpu.CompilerParams(needs_layout_passes=False),
        scratch_types=[
            pltpu.VMEM((n,), jnp.float32),
            pltpu.VMEM((n,), jnp.float32),
            pltpu.VMEM((_CH,), jnp.int32),
            pltpu.VMEM((_CH,), jnp.int32),
            pltpu.VMEM((_CH,), jnp.int32),
            pltpu.VMEM((16,), jnp.int32),
            pltpu.VMEM((16,), jnp.int32),
            pltpu.VMEM((16,), jnp.int32),
            pltpu.VMEM((_CH,), jnp.float32),
            pltpu.VMEM((_CH, 128), jnp.float32),
            pltpu.VMEM((_CH, 128), jnp.float32),
            pltpu.VMEM_SHARED((npad, 128), jnp.float32),
            pltpu.VMEM_SHARED((nd, 128), jnp.float32),
            pltpu.SemaphoreType.DMA,
        ],
    )
    def k(h_hbm, asv_hbm, adv_hbm, src_hbm, dst_hbm, rows_hbm, den_hbm,
          asv_v, adv_v, sidx, didx, didx2, sidx_t, didx_t, didx2_t,
          exv, rows, ob2, acc, dacc, sem):
        c = lax.axis_index("c")
        s = lax.axis_index("s")
        wid = s * _NC + c
        base = wid * epw

        pltpu.sync_copy(asv_hbm, asv_v)
        pltpu.sync_copy(adv_hbm, adv_v)

        def zrow(kk, _):
            for cc in range(8):
                rows[kk, pl.ds(16 * cc, 16)] = jnp.zeros((16,), jnp.float32)
                ob2[kk, pl.ds(16 * cc, 16)] = jnp.zeros((16,), jnp.float32)
            return 0
        lax.fori_loop(0, _CH, zrow, 0, unroll=4)

        for t in range(10):
            pltpu.sync_copy(rows.at[pl.ds(0, 64)],
                            acc.at[pl.ds(s * 640 + 64 * t, 64)])
        @pl.when(s == 0)
        def _():
            pltpu.sync_copy(rows.at[pl.ds(0, nd)], dacc)
        plsc.subcore_barrier()

        def do_chunk(off, kk, s_r, d_r, d2_r):
            pltpu.sync_copy(src_hbm.at[pl.ds(off, kk)], s_r)
            pltpu.sync_copy(dst_hbm.at[pl.ds(off, kk)], d_r)
            for g in range(kk // 16):
                sv = s_r[pl.ds(16 * g, 16)]
                dv = d_r[pl.ds(16 * g, 16)]
                a = plsc.load_gather(asv_v, [sv]) + plsc.load_gather(adv_v, [dv])
                a = jnp.where(a > 0, a, 0.2 * a)
                exg = jnp.exp(a)
                exv[pl.ds(16 * g, 16)] = exg
                d2_r[pl.ds(16 * g, 16)] = jnp.right_shift(dv, 7)
                ridx = lax.iota(jnp.int32, 16) + 16 * g
                plsc.store_scatter(ob2, [ridx, jnp.bitwise_and(dv, 127)], exg)
            if kk == _CH:
                pltpu.async_copy(h_hbm.at[s_r], rows, sem).wait()
            else:
                pltpu.async_copy(h_hbm.at[s_r], rows.at[pl.ds(0, kk)], sem).wait()

            def mrow(j, _):
                spl = plsc.load_gather(exv, [jnp.full((16,), 0, jnp.int32) + j])
                for pp in range(8):
                    rows[j, pl.ds(16 * pp, 16)] = rows[j, pl.ds(16 * pp, 16)] * spl
                return 0
            lax.fori_loop(0, kk, mrow, 0, unroll=4)
            if kk == _CH:
                pltpu.sync_copy(rows, acc.at[d_r], add=True)
                pltpu.sync_copy(ob2, dacc.at[d2_r], add=True)
            else:
                pltpu.sync_copy(rows.at[pl.ds(0, kk)], acc.at[d_r], add=True)
                pltpu.sync_copy(ob2.at[pl.ds(0, kk)], dacc.at[d2_r], add=True)
            zv = jnp.zeros((16,), jnp.float32)
            for g in range(kk // 16):
                dv = d_r[pl.ds(16 * g, 16)]
                ridx = lax.iota(jnp.int32, 16) + 16 * g
                plsc.store_scatter(ob2, [ridx, jnp.bitwise_and(dv, 127)], zv)

        def chunk_loop(j, _):
            do_chunk(base + j * _CH, _CH, sidx, didx, didx2)
            return 0
        lax.fori_loop(0, nfull, chunk_loop, 0)
        if rem:
            do_chunk(base + nfull * _CH, rem, sidx_t, didx_t, didx2_t)

        plsc.subcore_barrier()
        for t in range(5):
            r0 = s * 640 + 128 * t
            pltpu.sync_copy(acc.at[pl.ds(r0, 128)],
                            rows_hbm.at[c, pl.ds(r0, 128)])
        @pl.when(s < nd // 16)
        def _():
            pltpu.sync_copy(dacc.at[pl.ds(16 * s, 16)],
                            den_hbm.at[c, pl.ds(16 * s, 16)])

    return k(h, asv, adv, src, dst)


def _tc_post_a(acc, dvec, h, asv, adv, gatb, gnw, gnb, gnms):
    n = h.shape[0]

    def body(acc_ref, d_ref, h_ref, asv_ref, adv_ref, gatb_ref,
             gnw_ref, gnb_ref, gnms_ref, out_ref):
        als = asv_ref[...] + adv_ref[...]
        exs = jnp.exp(jnp.where(als > 0, als, 0.2 * als))
        raw = acc_ref[0] + acc_ref[1] + exs * h_ref[...]
        den = d_ref[0] + d_ref[1] + exs
        g = raw / (den + 1e-16) + gatb_ref[...]
        mean = jnp.mean(g, axis=0, keepdims=True)
        o = g - mean * gnms_ref[...]
        var = jnp.mean(o * o, axis=0, keepdims=True)
        out_ref[...] = gnw_ref[...] * o / jnp.sqrt(var + 1e-5) + gnb_ref[...]

    return pl.pallas_call(
        body,
        out_shape=jax.ShapeDtypeStruct((n, 128), jnp.float32),
    )(acc, dvec, h, asv, adv, gatb, gnw, gnb, gnms)


def _tc_post_b(g, xprev, mf, c1a, c1b_w, c1bias, c0a, c0b_w, c0bias,
               g2w, g2b, g2ms):
    n = g.shape[0]

    def body(g_ref, xp_ref, m_ref, c1a_ref, c1b_ref, c1bias_ref,
             c0a_ref, c0b_ref, c0bias_ref, g2w_ref, g2b_ref, g2ms_ref,
             out_ref):
        g = g_ref[...]
        xp = xp_ref[...]
        x1 = (jnp.dot(g, c1a_ref[...], preferred_element_type=jnp.float32)
              + jnp.dot(xp, c1b_ref[...], preferred_element_type=jnp.float32)
              + c1bias_ref[...])
        x0 = (jnp.dot(g, c0a_ref[...], preferred_element_type=jnp.float32)
              + jnp.dot(xp, c0b_ref[...], preferred_element_type=jnp.float32)
              + c0bias_ref[...])
        m = m_ref[...]
        x = m * (_ZR * x1 + (1 - _ZR) * x0) + (1 - m) * (_ZR * x0 + (1 - _ZR) * x1)
        mean = jnp.mean(x, axis=0, keepdims=True)
        o = x - mean * g2ms_ref[...]
        var = jnp.mean(o * o, axis=0, keepdims=True)
        out_ref[...] = g2w_ref[...] * o / jnp.sqrt(var + 1e-5) + g2b_ref[...]

    return pl.pallas_call(
        body,
        out_shape=jax.ShapeDtypeStruct((n, 128), jnp.float32),
    )(g, xprev, mf, c1a, c1b_w, c1bias, c0a, c0b_w, c0bias, g2w, g2b, g2ms)


def _sc_gather2(x, src, dst):
    n = x.shape[0]
    e = src.shape[0]
    epw = e // _NW
    nfull = epw // _CH
    rem = epw - nfull * _CH
    mesh = plsc.VectorSubcoreMesh(core_axis_name="c", subcore_axis_name="s")

    @functools.partial(
        pl.kernel,
        out_type=(
            jax.ShapeDtypeStruct((e, 128), jnp.float32),
            jax.ShapeDtypeStruct((e, 128), jnp.float32),
        ),
        mesh=mesh,
        compiler_params=pltpu.CompilerParams(needs_layout_passes=False),
        scratch_types=[
            pltpu.VMEM((_CH,), jnp.int32),
            pltpu.VMEM((_CH,), jnp.int32),
            pltpu.VMEM((16,), jnp.int32),
            pltpu.VMEM((16,), jnp.int32),
            pltpu.VMEM((_CH, 128), jnp.float32),
            pltpu.VMEM((_CH, 128), jnp.float32),
            pltpu.SemaphoreType.DMA,
            pltpu.SemaphoreType.DMA,
        ],
    )
    def k(x_hbm, src_hbm, dst_hbm, xs_hbm, xd_hbm,
          sidx, didx, sidx_t, didx_t, rs, rd, sem_s, sem_d):
        c = lax.axis_index("c")
        s = lax.axis_index("s")
        wid = s * _NC + c
        base = wid * epw

        def do_chunk(off, kk, s_r, d_r):
            pltpu.sync_copy(src_hbm.at[pl.ds(off, kk)], s_r)
            pltpu.sync_copy(dst_hbm.at[pl.ds(off, kk)], d_r)
            if kk == _CH:
                cps = pltpu.async_copy(x_hbm.at[s_r], rs, sem_s)
                cpd = pltpu.async_copy(x_hbm.at[d_r], rd, sem_d)
                cps.wait()
                cpd.wait()
                pltpu.sync_copy(rs, xs_hbm.at[pl.ds(off, kk)])
                pltpu.sync_copy(rd, xd_hbm.at[pl.ds(off, kk)])
            else:
                cps = pltpu.async_copy(x_hbm.at[s_r], rs.at[pl.ds(0, kk)], sem_s)
                cpd = pltpu.async_copy(x_hbm.at[d_r], rd.at[pl.ds(0, kk)], sem_d)
                cps.wait()
                cpd.wait()
                pltpu.sync_copy(rs.at[pl.ds(0, kk)], xs_hbm.at[pl.ds(off, kk)])
                pltpu.sync_copy(rd.at[pl.ds(0, kk)], xd_hbm.at[pl.ds(off, kk)])

        def chunk_loop(j, _):
            do_chunk(base + j * _CH, _CH, sidx, didx)
            return 0
        lax.fori_loop(0, nfull, chunk_loop, 0)
        if rem:
            do_chunk(base + nfull * _CH, rem, sidx_t, didx_t)

    return k(x, src, dst)


def _tc_edge(ea, xs, xd, mf3, ab3, wet, be, w1a, w1b, w1c, b1, w2t, b2,
             p1t, pb1, p2t, pb2, v1t, vb1, v2t, vb2, nb, bbe):
    e = ea.shape[0]

    def body(ea_ref, xs_ref, xd_ref, mf_ref, ab_ref, wet_ref, be_ref,
             w1a_ref, w1b_ref, w1c_ref, b1_ref, w2t_ref, b2_ref,
             p1t_ref, pb1_ref, p2t_ref, pb2_ref, v1t_ref, vb1_ref,
             v2t_ref, vb2_ref, trip_ref, lg_ref, st_ref):
        ee = _relu(jnp.dot(ea_ref[...], wet_ref[...],
                           preferred_element_type=jnp.float32) + be_ref[...])
        t = _relu(
            jnp.dot(xs_ref[...], w1a_ref[...], preferred_element_type=jnp.float32)
            + jnp.dot(ee, w1b_ref[...], preferred_element_type=jnp.float32)
            + jnp.dot(xd_ref[...], w1c_ref[...], preferred_element_type=jnp.float32)
            + b1_ref[...])
        tr = jnp.dot(t, w2t_ref[...], preferred_element_type=jnp.float32) + b2_ref[...]
        trip_ref[...] = tr
        hl = _relu(jnp.dot(tr, p1t_ref[...],
                           preferred_element_type=jnp.float32) + pb1_ref[...])
        lg = jnp.dot(hl, p2t_ref[...], preferred_element_type=jnp.float32) + pb2_ref[...]
        hv = _relu(jnp.dot(tr, v1t_ref[...],
                           preferred_element_type=jnp.float32) + vb1_ref[...])
        va = jnp.dot(hv, v2t_ref[...], preferred_element_type=jnp.float32) + vb2_ref[...]
        m = mf_ref[0]
        lgm = jnp.where(m > 0, lg, -1e9) + jnp.log(ab_ref[0] + 1e-10)
        avm = jnp.where(m > 0, va, 0.0)
        lg_ref[0] = lgm
        bm = jnp.max(lgm, keepdims=True)
        w = jnp.exp(lgm - bm)
        bs = jnp.sum(w, keepdims=True)
        bt = jnp.sum(w * lgm, keepdims=True)
        bv = jnp.sum(w * avm, keepdims=True)
        st_ref[0] = jnp.concatenate([bm, bs, bt, bv], axis=0)

    return pl.pallas_call(
        body,
        grid=(nb,),
        in_specs=[
            pl.BlockSpec((bbe, 128), lambda i: (i, 0)),
            pl.BlockSpec((bbe, 128), lambda i: (i, 0)),
            pl.BlockSpec((bbe, 128), lambda i: (i, 0)),
            pl.BlockSpec((1, bbe, 1), lambda i: (i, 0, 0)),
            pl.BlockSpec((1, bbe, 1), lambda i: (i, 0, 0)),
        ] + [pl.BlockSpec(w.shape, functools.partial(lambda r, i: (0,) * r,
                                                     len(w.shape)))
             for w in (wet, be, w1a, w1b, w1c, b1, w2t, b2,
                       p1t, pb1, p2t, pb2, v1t, vb1, v2t, vb2)],
        out_specs=[
            pl.BlockSpec((bbe, 128), lambda i: (i, 0)),
            pl.BlockSpec((1, bbe, 1), lambda i: (i, 0, 0)),
            pl.BlockSpec((1, 4, 1), lambda i: (i, 0, 0)),
        ],
        out_shape=[
            jax.ShapeDtypeStruct((e, 128), jnp.float32),
            jax.ShapeDtypeStruct((nb, bbe, 1), jnp.float32),
            jax.ShapeDtypeStruct((nb, 4, 1), jnp.float32),
        ],
    )(ea, xs, xd, mf3, ab3, wet, be, w1a, w1b, w1c, b1, w2t, b2,
      p1t, pb1, p2t, pb2, v1t, vb1, v2t, vb2)


def _tc_final(lg2, st2):
    nb, bbe = lg2.shape

    def body(lg_ref, st_ref, probs_ref, scal_ref):
        st = st_ref[...]
        bm = st[:, 0:1]
        m = jnp.max(bm, axis=0, keepdims=True)
        sc = jnp.exp(bm - m)
        s_ = jnp.sum(st[:, 1:2] * sc, axis=0, keepdims=True)
        t_ = jnp.sum(st[:, 2:3] * sc, axis=0, keepdims=True)
        v_ = jnp.sum(st[:, 3:4] * sc, axis=0, keepdims=True)
        logz = m + jnp.log(s_)
        ent = logz - t_ / s_
        sv = v_ / s_
        probs_ref[...] = jnp.exp(lg_ref[...] - logz[0, 0])
        scal_ref[...] = jnp.concatenate(
            [ent, sv, jnp.zeros((1, 6), jnp.float32)], axis=1)

    return pl.pallas_call(
        body,
        out_shape=(
            jax.ShapeDtypeStruct((nb, bbe), jnp.float32),
            jax.ShapeDtypeStruct((1, 8), jnp.float32),
        ),
    )(lg2, st2)


def kernel(x_, edge_index, edge_attr, question_embeddings, subgraph_mask,
           action_mask, action_bias, params):
    p = params
    n = x_.shape[0]
    e = edge_attr.shape[0]
    bbe = 2560 if e % 2560 == 0 else 64
    nb = e // bbe

    src = edge_index[0]
    dst = edge_index[1]
    mf = subgraph_mask.astype(jnp.float32).reshape(n, 1)
    mf3 = action_mask.astype(jnp.float32).reshape(nb, bbe, 1)
    ab3 = action_bias.reshape(nb, bbe, 1)

    def t(w):
        return w.T

    def r1(v):
        return v.reshape(1, -1)

    xm = _tc_prep(x_, question_embeddings,
                  t(p['node_w']), r1(p['node_b']),
                  t(p['q_w']), r1(p['q_b']),
                  t(p['mix_w'][:, :128]), t(p['mix_w'][:, 128:]),
                  r1(p['mix_b']))

    xcur = xm
    for l in range(2):
        h, asv, adv = _tc_pre(
            xcur, mf,
            t(p[f'c{l}_t1_w']), r1(p[f'c{l}_t1_b']),
            t(p[f'c{l}_t0_w']), r1(p[f'c{l}_t0_b']),
            t(p[f'c{l}_gat_w']),
            p[f'c{l}_att_src'].reshape(128, 1),
            p[f'c{l}_att_dst'].reshape(128, 1))
        accp, denp = _sc_gat(h, asv.reshape(n), adv.reshape(n), src, dst)
        acc = accp[:, :n, :]
        dvec = denp.reshape(2, -1)[:, :n].reshape(2, n, 1)
        g = _tc_post_a(
            acc, dvec, h, asv, adv,
            r1(p[f'c{l}_gat_b']),
            r1(p[f'c{l}_gn_w']), r1(p[f'c{l}_gn_b']), r1(p[f'c{l}_gn_ms']))
        xcur = _tc_post_b(
            g, xcur, mf,
            t(p[f'c{l}_c1_w'][:, :128]), t(p[f'c{l}_c1_w'][:, 128:]),
            r1(p[f'c{l}_c1_b']),
            t(p[f'c{l}_c0_w'][:, :128]), t(p[f'c{l}_c0_w'][:, 128:]),
            r1(p[f'c{l}_c0_b']),
            r1(p[f'gn{l}_w']), r1(p[f'gn{l}_b']), r1(p[f'gn{l}_ms']))

    xs, xd = _sc_gather2(xcur, src, dst)

    trip, lg3, st3 = _tc_edge(
        edge_attr, xs, xd, mf3, ab3,
        t(p['edge_w']), r1(p['edge_b']),
        t(p['tm1_w'][:, :128]), t(p['tm1_w'][:, 128:256]),
        t(p['tm1_w'][:, 256:]), r1(p['tm1_b']),
        t(p['tm2_w']), r1(p['tm2_b']),
        t(p['ph1_w']), r1(p['ph1_b']),
        t(p['ph2_w']), r1(p['ph2_b']),
        t(p['vh1_w']), r1(p['vh1_b']),
        t(p['vh2_w']), r1(p['vh2_b']),
        nb, bbe)

    probs2, scal = _tc_final(lg3.reshape(nb, bbe), st3.reshape(nb, 4))
    probs = probs2.reshape(e)
    entropy = scal[0, 0]
    state_value = scal[0, 1]
    return probs, state_value, trip, entropy


# speedup vs baseline: 12.0239x; 1.0673x over previous
import functools

import jax
import jax.numpy as jnp
from jax import lax
from jax.experimental import pallas as pl
from jax.experimental.pallas import tpu as pltpu
from jax.experimental.pallas import tpu_sc as plsc

_ZR = 0.8
_NC = 2
_NS = 16
_NW = _NC * _NS
_CH = 128


def _relu(x):
    return jnp.maximum(x, 0.0)


def _tc_prep(x_, qe, nwt, nb, qwt, qb, mxt, mqt, mb):
    n, d = x_.shape

    def body(x_ref, qe_ref, nwt_ref, nb_ref, qwt_ref, qb_ref, mxt_ref,
             mqt_ref, mb_ref, out_ref):
        x = _relu(jnp.dot(x_ref[...], nwt_ref[...],
                          preferred_element_type=jnp.float32) + nb_ref[...])
        q = _relu(jnp.dot(qe_ref[...], qwt_ref[...],
                          preferred_element_type=jnp.float32) + qb_ref[...])
        qc = jnp.dot(q, mqt_ref[...], preferred_element_type=jnp.float32)
        out_ref[...] = _relu(
            jnp.dot(x, mxt_ref[...], preferred_element_type=jnp.float32)
            + qc + mb_ref[...])

    return pl.pallas_call(
        body,
        out_shape=jax.ShapeDtypeStruct((n, 128), jnp.float32),
    )(x_, qe, nwt, nb, qwt, qb, mxt, mqt, mb)


def _tc_pre(x, mf, t1t, t1b, t0t, t0b, gt, a_s, a_d):
    n = x.shape[0]

    def body(x_ref, m_ref, t1t_ref, t1b_ref, t0t_ref, t0b_ref, gt_ref,
             as_ref, ad_ref, h_out, asv_out, adv_out):
        x = x_ref[...]
        t1 = _relu(jnp.dot(x, t1t_ref[...],
                           preferred_element_type=jnp.float32) + t1b_ref[...])
        t0 = _relu(jnp.dot(x, t0t_ref[...],
                           preferred_element_type=jnp.float32) + t0b_ref[...])
        m = m_ref[...]
        xm = m * (_ZR * t1 + (1 - _ZR) * t0) + (1 - m) * (_ZR * t0 + (1 - _ZR) * t1)
        h = jnp.dot(xm, gt_ref[...], preferred_element_type=jnp.float32)
        h_out[...] = h
        asv_out[...] = jnp.dot(h, as_ref[...], preferred_element_type=jnp.float32)
        adv_out[...] = jnp.dot(h, ad_ref[...], preferred_element_type=jnp.float32)

    return pl.pallas_call(
        body,
        out_shape=(
            jax.ShapeDtypeStruct((n, 128), jnp.float32),
            jax.ShapeDtypeStruct((n, 1), jnp.float32),
            jax.ShapeDtypeStruct((n, 1), jnp.float32),
        ),
    )(x, mf, t1t, t1b, t0t, t0b, gt, a_s, a_d)


def _sc_gat(h, asv, adv, src, dst):
    n = h.shape[0]
    e = src.shape[0]
    epw = e // _NW
    nfull = epw // _CH
    rem = epw - nfull * _CH
    npad = _NS * 640
    nd = npad // 128
    assert n <= npad
    mesh = plsc.VectorSubcoreMesh(core_axis_name="c", subcore_axis_name="s")

    @functools.partial(
        pl.kernel,
        out_type=(
            jax.ShapeDtypeStruct((_NC, npad, 128), jnp.float32),
            jax.ShapeDtypeStruct((_NC, nd, 128), jnp.float32),
        ),
        mesh=mesh,
        compiler_params=pltpu.CompilerParams(needs_layout_passes=False),
        scratch_types=[
            pltpu.VMEM((n,), jnp.float32),
            pltpu.VMEM((n,), jnp.float32),
            pltpu.VMEM((_CH,), jnp.int32),
            pltpu.VMEM((_CH,), jnp.int32),
            pltpu.VMEM((16,), jnp.int32),
            pltpu.VMEM((16,), jnp.int32),
            pltpu.VMEM((_CH,), jnp.float32),
            pltpu.VMEM((_CH, 128), jnp.float32),
            pltpu.VMEM((80, 128), jnp.float32),
            pltpu.VMEM((80,), jnp.int32),
            pltpu.VMEM((16,), jnp.int32),
            pltpu.VMEM((16,), jnp.float32),
            pltpu.VMEM_SHARED((npad, 128), jnp.float32),
            pltpu.VMEM_SHARED((nd, 128), jnp.float32),
            pltpu.SemaphoreType.DMA,
        ],
    )
    def k(h_hbm, asv_hbm, adv_hbm, src_hbm, dst_hbm, rows_hbm, den_hbm,
          asv_v, adv_v, sidx, didx, sidx_t, didx_t,
          exv, rows, denloc, rowidx, ksc, ssc, acc, dacc, sem):
        c = lax.axis_index("c")
        s = lax.axis_index("s")
        wid = s * _NC + c
        base = wid * epw
        io = lax.iota(jnp.int32, 16)

        pltpu.sync_copy(asv_hbm, asv_v)
        pltpu.sync_copy(adv_hbm, adv_v)

        zv = jnp.zeros((16,), jnp.float32)
        def zrow(kk, _):
            for cc in range(8):
                rows[kk, pl.ds(16 * cc, 16)] = zv
            return 0
        lax.fori_loop(0, _CH, zrow, 0, unroll=4)
        def zden(kk, _):
            for cc in range(8):
                denloc[kk, pl.ds(16 * cc, 16)] = zv
            return 0
        lax.fori_loop(0, 80, zden, 0, unroll=4)
        for t in range(5):
            rowidx[pl.ds(16 * t, 16)] = io + 16 * t

        for t in range(5):
            pltpu.sync_copy(rows, acc.at[pl.ds(s * 640 + 128 * t, 128)])
        @pl.when(s == 0)
        def _():
            pltpu.sync_copy(rows.at[pl.ds(0, nd)], dacc)
        plsc.subcore_barrier()

        def do_chunk(off, kk, s_r, d_r):
            pltpu.sync_copy(src_hbm.at[pl.ds(off, kk)], s_r)
            pltpu.sync_copy(dst_hbm.at[pl.ds(off, kk)], d_r)
            for g in range(kk // 16):
                sv = s_r[pl.ds(16 * g, 16)]
                dv = d_r[pl.ds(16 * g, 16)]
                a = plsc.load_gather(asv_v, [sv]) + plsc.load_gather(adv_v, [dv])
                a = jnp.where(a > 0, a, 0.2 * a)
                exg = jnp.exp(a)
                exv[pl.ds(16 * g, 16)] = exg
                sk, sv2 = plsc.sort_key_val(dv, exg)
                ksc[...] = sk
                cs = plsc.cumsum(sv2)
                ssc[...] = cs
                prevk = plsc.load_gather(ksc, [jnp.maximum(io - 1, 0)])
                is_start = jnp.logical_or(io == 0, sk != prevk)
                sid = plsc.cummax(jnp.where(is_start, io, 0))
                sprev = plsc.load_gather(ssc, [jnp.maximum(sid - 1, 0)])
                runsum = cs - jnp.where(sid > 0, sprev, 0.0)
                nextk = plsc.load_gather(ksc, [jnp.minimum(io + 1, 15)])
                is_end = jnp.logical_or(io == 15, sk != nextk)
                plsc.addupdate_scatter(
                    denloc,
                    [jnp.right_shift(sk, 7), jnp.bitwise_and(sk, 127)],
                    runsum, mask=is_end)
            if kk == _CH:
                pltpu.async_copy(h_hbm.at[s_r], rows, sem).wait()
            else:
                pltpu.async_copy(h_hbm.at[s_r], rows.at[pl.ds(0, kk)], sem).wait()

            def mrow(j, _):
                spl = plsc.load_gather(exv, [jnp.full((16,), 0, jnp.int32) + j])
                for pp in range(8):
                    rows[j, pl.ds(16 * pp, 16)] = rows[j, pl.ds(16 * pp, 16)] * spl
                return 0
            lax.fori_loop(0, kk, mrow, 0, unroll=4)
            if kk == _CH:
                pltpu.sync_copy(rows, acc.at[d_r], add=True)
            else:
                pltpu.sync_copy(rows.at[pl.ds(0, kk)], acc.at[d_r], add=True)

        def chunk_loop(j, _):
            do_chunk(base + j * _CH, _CH, sidx, didx)
            return 0
        lax.fori_loop(0, nfull, chunk_loop, 0)
        if rem:
            do_chunk(base + nfull * _CH, rem, sidx_t, didx_t)

        pltpu.sync_copy(denloc, dacc.at[rowidx], add=True)

        plsc.subcore_barrier()
        for t in range(5):
            r0 = s * 640 + 128 * t
            pltpu.sync_copy(acc.at[pl.ds(r0, 128)],
                            rows_hbm.at[c, pl.ds(r0, 128)])
        @pl.when(s < nd // 16)
        def _():
            pltpu.sync_copy(dacc.at[pl.ds(16 * s, 16)],
                            den_hbm.at[c, pl.ds(16 * s, 16)])

    return k(h, asv, adv, src, dst)


def _tc_post_a(acc, dvec, h, asv, adv, gatb, gnw, gnb, gnms):
    n = h.shape[0]

    def body(acc_ref, d_ref, h_ref, asv_ref, adv_ref, gatb_ref,
             gnw_ref, gnb_ref, gnms_ref, out_ref):
        als = asv_ref[...] + adv_ref[...]
        exs = jnp.exp(jnp.where(als > 0, als, 0.2 * als))
        raw = acc_ref[0] + acc_ref[1] + exs * h_ref[...]
        den = d_ref[0] + d_ref[1] + exs
        g = raw / (den + 1e-16) + gatb_ref[...]
        mean = jnp.mean(g, axis=0, keepdims=True)
        o = g - mean * gnms_ref[...]
        var = jnp.mean(o * o, axis=0, keepdims=True)
        out_ref[...] = gnw_ref[...] * o / jnp.sqrt(var + 1e-5) + gnb_ref[...]

    return pl.pallas_call(
        body,
        out_shape=jax.ShapeDtypeStruct((n, 128), jnp.float32),
    )(acc, dvec, h, asv, adv, gatb, gnw, gnb, gnms)


def _tc_post_b(g, xprev, mf, c1a, c1b_w, c1bias, c0a, c0b_w, c0bias,
               g2w, g2b, g2ms):
    n = g.shape[0]

    def body(g_ref, xp_ref, m_ref, c1a_ref, c1b_ref, c1bias_ref,
             c0a_ref, c0b_ref, c0bias_ref, g2w_ref, g2b_ref, g2ms_ref,
             out_ref):
        g = g_ref[...]
        xp = xp_ref[...]
        x1 = (jnp.dot(g, c1a_ref[...], preferred_element_type=jnp.float32)
              + jnp.dot(xp, c1b_ref[...], preferred_element_type=jnp.float32)
              + c1bias_ref[...])
        x0 = (jnp.dot(g, c0a_ref[...], preferred_element_type=jnp.float32)
              + jnp.dot(xp, c0b_ref[...], preferred_element_type=jnp.float32)
              + c0bias_ref[...])
        m = m_ref[...]
        x = m * (_ZR * x1 + (1 - _ZR) * x0) + (1 - m) * (_ZR * x0 + (1 - _ZR) * x1)
        mean = jnp.mean(x, axis=0, keepdims=True)
        o = x - mean * g2ms_ref[...]
        var = jnp.mean(o * o, axis=0, keepdims=True)
        out_ref[...] = g2w_ref[...] * o / jnp.sqrt(var + 1e-5) + g2b_ref[...]

    return pl.pallas_call(
        body,
        out_shape=jax.ShapeDtypeStruct((n, 128), jnp.float32),
    )(g, xprev, mf, c1a, c1b_w, c1bias, c0a, c0b_w, c0bias, g2w, g2b, g2ms)


def _sc_gather2(x, src, dst):
    n = x.shape[0]
    e = src.shape[0]
    epw = e // _NW
    CH = 128
    nfull = epw // CH
    rem = epw - nfull * CH
    half = nfull // 2
    assert nfull == 2 * half
    mesh = plsc.VectorSubcoreMesh(core_axis_name="c", subcore_axis_name="s")

    @functools.partial(
        pl.kernel,
        out_type=(
            jax.ShapeDtypeStruct((e, 128), jnp.float32),
            jax.ShapeDtypeStruct((e, 128), jnp.float32),
        ),
        mesh=mesh,
        compiler_params=pltpu.CompilerParams(needs_layout_passes=False),
        scratch_types=(
            [pltpu.VMEM((CH,), jnp.int32)] * 4
            + [pltpu.VMEM((16,), jnp.int32)] * 2
            + [pltpu.VMEM((CH, 128), jnp.float32)] * 4
            + [pltpu.SemaphoreType.DMA] * 12
        ),
    )
    def k(x_hbm, src_hbm, dst_hbm, xs_hbm, xd_hbm,
          si0, di0, si1, di1, sit, dit, rs0, rd0, rs1, rd1,
          ssi0, sdi0, ssi1, sdi1, sgs0, sgd0, sgs1, sgd1,
          sws0, swd0, sws1, swd1):
        c = lax.axis_index("c")
        s = lax.axis_index("s")
        wid = s * _NC + c
        base = wid * epw

        sets = ((si0, di0, rs0, rd0, ssi0, sdi0, sgs0, sgd0, sws0, swd0),
                (si1, di1, rs1, rd1, ssi1, sdi1, sgs1, sgd1, sws1, swd1))

        def idx_start(off, st):
            si, di = st[0], st[1]
            pltpu.async_copy(src_hbm.at[pl.ds(off, CH)], si, st[4])
            pltpu.async_copy(dst_hbm.at[pl.ds(off, CH)], di, st[5])

        def idx_wait(st):
            pltpu.make_async_copy(src_hbm.at[pl.ds(0, CH)], st[0], st[4]).wait()
            pltpu.make_async_copy(dst_hbm.at[pl.ds(0, CH)], st[1], st[5]).wait()

        def write_wait(st):
            pltpu.make_async_copy(st[2], xs_hbm.at[pl.ds(0, CH)], st[8]).wait()
            pltpu.make_async_copy(st[3], xd_hbm.at[pl.ds(0, CH)], st[9]).wait()

        def process(off, st, j, first):
            @pl.when(jnp.logical_not(first))
            def _():
                write_wait(st)
            idx_wait(st)
            cs = pltpu.async_copy(x_hbm.at[st[0]], st[2], st[6])
            cd = pltpu.async_copy(x_hbm.at[st[1]], st[3], st[7])
            cs.wait()
            cd.wait()
            pltpu.async_copy(st[2], xs_hbm.at[pl.ds(off, CH)], st[8])
            pltpu.async_copy(st[3], xd_hbm.at[pl.ds(off, CH)], st[9])

        idx_start(base, sets[0])

        def loop(j, _):
            a_off = base + (2 * j) * CH
            b_off = base + (2 * j + 1) * CH
            idx_start(b_off, sets[1])
            process(a_off, sets[0], j, j == 0)
            @pl.when(j < half - 1)
            def _():
                idx_start(a_off + 2 * CH, sets[0])
            process(b_off, sets[1], j, j == 0)
            return 0
        lax.fori_loop(0, half, loop, 0)
        write_wait(sets[0])
        write_wait(sets[1])

        if rem:
            off = base + nfull * CH
            pltpu.sync_copy(src_hbm.at[pl.ds(off, rem)], sit)
            pltpu.sync_copy(dst_hbm.at[pl.ds(off, rem)], dit)
            cs = pltpu.async_copy(x_hbm.at[sit], rs0.at[pl.ds(0, rem)], sgs0)
            cd = pltpu.async_copy(x_hbm.at[dit], rd0.at[pl.ds(0, rem)], sgd0)
            cs.wait()
            cd.wait()
            pltpu.sync_copy(rs0.at[pl.ds(0, rem)], xs_hbm.at[pl.ds(off, rem)])
            pltpu.sync_copy(rd0.at[pl.ds(0, rem)], xd_hbm.at[pl.ds(off, rem)])

    return k(x, src, dst)


def _tc_edge(ea, xs, xd, mf3, ab3, wet, be, w1a, w1b, w1c, b1, w2t, b2,
             p1t, pb1, p2t, pb2, v1t, vb1, v2t, vb2, nb, bbe):
    e = ea.shape[0]

    def body(ea_ref, xs_ref, xd_ref, mf_ref, ab_ref, wet_ref, be_ref,
             w1a_ref, w1b_ref, w1c_ref, b1_ref, w2t_ref, b2_ref,
             p1t_ref, pb1_ref, p2t_ref, pb2_ref, v1t_ref, vb1_ref,
             v2t_ref, vb2_ref, trip_ref, lg_ref, st_ref):
        ee = _relu(jnp.dot(ea_ref[...], wet_ref[...],
                           preferred_element_type=jnp.float32) + be_ref[...])
        t = _relu(
            jnp.dot(xs_ref[...], w1a_ref[...], preferred_element_type=jnp.float32)
            + jnp.dot(ee, w1b_ref[...], preferred_element_type=jnp.float32)
            + jnp.dot(xd_ref[...], w1c_ref[...], preferred_element_type=jnp.float32)
            + b1_ref[...])
        tr = jnp.dot(t, w2t_ref[...], preferred_element_type=jnp.float32) + b2_ref[...]
        trip_ref[...] = tr
        hl = _relu(jnp.dot(tr, p1t_ref[...],
                           preferred_element_type=jnp.float32) + pb1_ref[...])
        lg = jnp.dot(hl, p2t_ref[...], preferred_element_type=jnp.float32) + pb2_ref[...]
        hv = _relu(jnp.dot(tr, v1t_ref[...],
                           preferred_element_type=jnp.float32) + vb1_ref[...])
        va = jnp.dot(hv, v2t_ref[...], preferred_element_type=jnp.float32) + vb2_ref[...]
        m = mf_ref[0]
        lgm = jnp.where(m > 0, lg, -1e9) + jnp.log(ab_ref[0] + 1e-10)
        avm = jnp.where(m > 0, va, 0.0)
        lg_ref[0] = lgm
        bm = jnp.max(lgm, keepdims=True)
        w = jnp.exp(lgm - bm)
        bs = jnp.sum(w, keepdims=True)
        bt = jnp.sum(w * lgm, keepdims=True)
        bv = jnp.sum(w * avm, keepdims=True)
        st_ref[0] = jnp.concatenate([bm, bs, bt, bv], axis=0)

    return pl.pallas_call(
        body,
        grid=(nb,),
        in_specs=[
            pl.BlockSpec((bbe, 128), lambda i: (i, 0)),
            pl.BlockSpec((bbe, 128), lambda i: (i, 0)),
            pl.BlockSpec((bbe, 128), lambda i: (i, 0)),
            pl.BlockSpec((1, bbe, 1), lambda i: (i, 0, 0)),
            pl.BlockSpec((1, bbe, 1), lambda i: (i, 0, 0)),
        ] + [pl.BlockSpec(w.shape, functools.partial(lambda r, i: (0,) * r,
                                                     len(w.shape)))
             for w in (wet, be, w1a, w1b, w1c, b1, w2t, b2,
                       p1t, pb1, p2t, pb2, v1t, vb1, v2t, vb2)],
        out_specs=[
            pl.BlockSpec((bbe, 128), lambda i: (i, 0)),
            pl.BlockSpec((1, bbe, 1), lambda i: (i, 0, 0)),
            pl.BlockSpec((1, 4, 1), lambda i: (i, 0, 0)),
        ],
        out_shape=[
            jax.ShapeDtypeStruct((e, 128), jnp.float32),
            jax.ShapeDtypeStruct((nb, bbe, 1), jnp.float32),
            jax.ShapeDtypeStruct((nb, 4, 1), jnp.float32),
        ],
    )(ea, xs, xd, mf3, ab3, wet, be, w1a, w1b, w1c, b1, w2t, b2,
      p1t, pb1, p2t, pb2, v1t, vb1, v2t, vb2)


def _tc_final(lg2, st2):
    nb, bbe = lg2.shape

    def body(lg_ref, st_ref, probs_ref, scal_ref):
        st = st_ref[...]
        bm = st[:, 0:1]
        m = jnp.max(bm, axis=0, keepdims=True)
        sc = jnp.exp(bm - m)
        s_ = jnp.sum(st[:, 1:2] * sc, axis=0, keepdims=True)
        t_ = jnp.sum(st[:, 2:3] * sc, axis=0, keepdims=True)
        v_ = jnp.sum(st[:, 3:4] * sc, axis=0, keepdims=True)
        logz = m + jnp.log(s_)
        ent = logz - t_ / s_
        sv = v_ / s_
        probs_ref[...] = jnp.exp(lg_ref[...] - logz[0, 0])
        scal_ref[...] = jnp.concatenate(
            [ent, sv, jnp.zeros((1, 6), jnp.float32)], axis=1)

    return pl.pallas_call(
        body,
        out_shape=(
            jax.ShapeDtypeStruct((nb, bbe), jnp.float32),
            jax.ShapeDtypeStruct((1, 8), jnp.float32),
        ),
    )(lg2, st2)


def kernel(x_, edge_index, edge_attr, question_embeddings, subgraph_mask,
           action_mask, action_bias, params):
    p = params
    n = x_.shape[0]
    e = edge_attr.shape[0]
    bbe = 2560 if e % 2560 == 0 else 64
    nb = e // bbe

    src = edge_index[0]
    dst = edge_index[1]
    mf = subgraph_mask.astype(jnp.float32).reshape(n, 1)
    mf3 = action_mask.astype(jnp.float32).reshape(nb, bbe, 1)
    ab3 = action_bias.reshape(nb, bbe, 1)

    def t(w):
        return w.T

    def r1(v):
        return v.reshape(1, -1)

    xm = _tc_prep(x_, question_embeddings,
                  t(p['node_w']), r1(p['node_b']),
                  t(p['q_w']), r1(p['q_b']),
                  t(p['mix_w'][:, :128]), t(p['mix_w'][:, 128:]),
                  r1(p['mix_b']))

    xcur = xm
    for l in range(2):
        h, asv, adv = _tc_pre(
            xcur, mf,
            t(p[f'c{l}_t1_w']), r1(p[f'c{l}_t1_b']),
            t(p[f'c{l}_t0_w']), r1(p[f'c{l}_t0_b']),
            t(p[f'c{l}_gat_w']),
            p[f'c{l}_att_src'].reshape(128, 1),
            p[f'c{l}_att_dst'].reshape(128, 1))
        accp, denp = _sc_gat(h, asv.reshape(n), adv.reshape(n), src, dst)
        acc = accp[:, :n, :]
        dvec = denp.reshape(2, -1)[:, :n].reshape(2, n, 1)
        g = _tc_post_a(
            acc, dvec, h, asv, adv,
            r1(p[f'c{l}_gat_b']),
            r1(p[f'c{l}_gn_w']), r1(p[f'c{l}_gn_b']), r1(p[f'c{l}_gn_ms']))
        xcur = _tc_post_b(
            g, xcur, mf,
            t(p[f'c{l}_c1_w'][:, :128]), t(p[f'c{l}_c1_w'][:, 128:]),
            r1(p[f'c{l}_c1_b']),
            t(p[f'c{l}_c0_w'][:, :128]), t(p[f'c{l}_c0_w'][:, 128:]),
            r1(p[f'c{l}_c0_b']),
            r1(p[f'gn{l}_w']), r1(p[f'gn{l}_b']), r1(p[f'gn{l}_ms']))

    xs, xd = _sc_gather2(xcur, src, dst)

    trip, lg3, st3 = _tc_edge(
        edge_attr, xs, xd, mf3, ab3,
        t(p['edge_w']), r1(p['edge_b']),
        t(p['tm1_w'][:, :128]), t(p['tm1_w'][:, 128:256]),
        t(p['tm1_w'][:, 256:]), r1(p['tm1_b']),
        t(p['tm2_w']), r1(p['tm2_b']),
        t(p['ph1_w']), r1(p['ph1_b']),
        t(p['ph2_w']), r1(p['ph2_b']),
        t(p['vh1_w']), r1(p['vh1_b']),
        t(p['vh2_w']), r1(p['vh2_b']),
        nb, bbe)

    probs2, scal = _tc_final(lg3.reshape(nb, bbe), st3.reshape(nb, 4))
    probs = probs2.reshape(e)
    entropy = scal[0, 0]
    state_value = scal[0, 1]
    return probs, state_value, trip, entropy
